# baseline (speedup 1.0000x reference)
import functools

import jax
import jax.numpy as jnp
from jax import lax
from jax.experimental import pallas as pl
from jax.experimental.pallas import tpu as pltpu
from jax.experimental.pallas import tpu_sc as plsc

ROWS = 8192
D = 768
NUM_WORKERS = 32
ROWS_PER_W = ROWS // NUM_WORKERS
CHUNK = 16
NCHUNK = ROWS_PER_W // CHUNK
NBUF = 10

_mesh = plsc.VectorSubcoreMesh(core_axis_name="c", subcore_axis_name="s")


@functools.partial(
    pl.kernel,
    out_type=jax.ShapeDtypeStruct((ROWS, D), jnp.float32),
    mesh=_mesh,
    scratch_types=(
        [pltpu.VMEM((CHUNK, D), jnp.float32) for _ in range(NBUF)]
        + [pltpu.SemaphoreType.DMA, pltpu.SemaphoreType.DMA]
    ),
)
def _pe_linear_gather(pe_hbm, out_hbm, *scratch):
    bufs = scratch[:NBUF]
    in_sem, out_sem = scratch[NBUF], scratch[NBUF + 1]
    wid = lax.axis_index("s") * 2 + lax.axis_index("c")
    base = wid * ROWS_PER_W

    def slab(i):
        return pl.ds(base + i * CHUNK, CHUNK)

    def load(i):
        return pltpu.async_copy(pe_hbm.at[slab(i)], bufs[i % NBUF], in_sem)

    def store(i):
        return pltpu.async_copy(bufs[i % NBUF], out_hbm.at[slab(i)], out_sem)

    loads = [load(i) for i in range(NBUF)]
    stores = []
    for i in range(NCHUNK):
        loads[i].wait()
        stores.append(store(i))
        if i + NBUF < NCHUNK:
            stores[i].wait()
            loads.append(load(i + NBUF))
    for i in range(max(0, NCHUNK - NBUF), NCHUNK):
        stores[i].wait()


def kernel(x, pe):
    del x
    return _pe_linear_gather(pe)


# speedup vs baseline: 1.0137x; 1.0137x over previous
import functools

import jax
import jax.numpy as jnp
from jax import lax
from jax.experimental import pallas as pl
from jax.experimental.pallas import tpu as pltpu
from jax.experimental.pallas import tpu_sc as plsc

ROWS = 8192
D = 768
NUM_WORKERS = 32
ROWS_PER_W = ROWS // NUM_WORKERS
CHUNK = 32
NCHUNK = ROWS_PER_W // CHUNK
NBUF = 5

_mesh = plsc.VectorSubcoreMesh(core_axis_name="c", subcore_axis_name="s")


@functools.partial(
    pl.kernel,
    out_type=jax.ShapeDtypeStruct((ROWS, D), jnp.float32),
    mesh=_mesh,
    scratch_types=(
        [pltpu.VMEM((CHUNK, D), jnp.float32) for _ in range(NBUF)]
        + [pltpu.SemaphoreType.DMA, pltpu.SemaphoreType.DMA]
    ),
)
def _pe_linear_gather(pe_hbm, out_hbm, *scratch):
    bufs = scratch[:NBUF]
    in_sem, out_sem = scratch[NBUF], scratch[NBUF + 1]
    wid = lax.axis_index("c") * 16 + lax.axis_index("s")
    base = wid * ROWS_PER_W

    def slab(i):
        return pl.ds(base + i * CHUNK, CHUNK)

    def load(i):
        return pltpu.async_copy(pe_hbm.at[slab(i)], bufs[i % NBUF], in_sem)

    def store(i):
        return pltpu.async_copy(bufs[i % NBUF], out_hbm.at[slab(i)], out_sem)

    loads = [load(i) for i in range(NBUF)]
    stores = []
    for i in range(NCHUNK):
        loads[i].wait()
        stores.append(store(i))
        if i + NBUF < NCHUNK:
            stores[i].wait()
            loads.append(load(i + NBUF))
    for i in range(max(0, NCHUNK - NBUF), NCHUNK):
        stores[i].wait()


def kernel(x, pe):
    del x
    return _pe_linear_gather(pe)


# speedup vs baseline: 1.0239x; 1.0100x over previous
import functools

import jax
import jax.numpy as jnp
from jax import lax
from jax.experimental import pallas as pl
from jax.experimental.pallas import tpu as pltpu
from jax.experimental.pallas import tpu_sc as plsc

ROWS = 8192
D = 768
NUM_WORKERS = 32
ROWS_PER_W = ROWS // NUM_WORKERS
CHUNK = 32
NCHUNK = ROWS_PER_W // CHUNK
NBUF = 5

_mesh = plsc.VectorSubcoreMesh(core_axis_name="c", subcore_axis_name="s")


@functools.partial(
    pl.kernel,
    out_type=jax.ShapeDtypeStruct((ROWS, D), jnp.float32),
    mesh=_mesh,
    scratch_types=(
        [pltpu.VMEM((CHUNK, D), jnp.float32) for _ in range(NBUF)]
        + [pltpu.SemaphoreType.DMA for _ in range(2 * NCHUNK)]
    ),
)
def _pe_linear_gather(pe_hbm, out_hbm, *scratch):
    bufs = scratch[:NBUF]
    in_sems = scratch[NBUF : NBUF + NCHUNK]
    out_sems = scratch[NBUF + NCHUNK :]
    wid = lax.axis_index("s") * 2 + lax.axis_index("c")
    base = wid * ROWS_PER_W

    def slab(i):
        return pl.ds(base + i * CHUNK, CHUNK)

    def load(i):
        return pltpu.async_copy(pe_hbm.at[slab(i)], bufs[i % NBUF], in_sems[i])

    def store(i):
        return pltpu.async_copy(bufs[i % NBUF], out_hbm.at[slab(i)], out_sems[i])

    loads = [load(i) for i in range(NBUF)]
    stores = []
    for i in range(NCHUNK):
        loads[i].wait()
        stores.append(store(i))
        if i + NBUF < NCHUNK:
            stores[i].wait()
            loads.append(load(i + NBUF))
    for i in range(max(0, NCHUNK - NBUF), NCHUNK):
        stores[i].wait()


def kernel(x, pe):
    del x
    return _pe_linear_gather(pe)
